# Initial kernel scaffold; baseline (speedup 1.0000x reference)
import jax
import jax.numpy as jnp
from jax.experimental import pallas as pl


def kernel(depth, height, width, batch_size, d_emb, h_emb, w_emb):
    raise NotImplementedError("write your pallas kernel here")


# speedup vs baseline: 10.4037x; 10.4037x over previous
import jax
import jax.numpy as jnp
from jax import lax
from jax.experimental import pallas as pl
from jax.experimental.pallas import tpu as pltpu
from jax.experimental.pallas import tpu_sc as plsc

_D = _H = _W = 64
_EMB = 64
_ROW = 3 * _EMB
_NV = _EMB // 16


def _body(d_hbm, h_hbm, w_hbm, out_hbm, tab_d, tab_h, tab_w, blk0, blk1,
          sem0, sem1):
    wid = lax.axis_index("s") * 2 + lax.axis_index("c")

    pltpu.sync_copy(d_hbm.at[pl.ds(0, _D)], tab_d)
    pltpu.sync_copy(h_hbm.at[pl.ds(0, _H)], tab_h)
    pltpu.sync_copy(w_hbm.at[pl.ds(0, _W)], tab_w)

    blks = (blk0, blk1)
    sems = (sem0, sem1)

    def fill_w(r, carry):
        for k in range(_NV):
            v = tab_w[r, pl.ds(16 * k, 16)]
            blk0[r, pl.ds(2 * _EMB + 16 * k, 16)] = v
            blk1[r, pl.ds(2 * _EMB + 16 * k, 16)] = v
        return carry
    lax.fori_loop(0, _W, fill_w, 0)

    def fill_h(h, blk):
        hv = [tab_h[h, pl.ds(16 * k, 16)] for k in range(_NV)]
        def body(r, carry):
            for k in range(_NV):
                blk[r, pl.ds(_EMB + 16 * k, 16)] = hv[k]
            return carry
        lax.fori_loop(0, _W, body, 0)

    for dd in range(2):
        d = wid * 2 + dd
        dv = [tab_d[d, pl.ds(16 * k, 16)] for k in range(_NV)]

        def fill_d(r, carry):
            for k in range(_NV):
                blk0[r, pl.ds(16 * k, 16)] = dv[k]
                blk1[r, pl.ds(16 * k, 16)] = dv[k]
            return carry
        lax.fori_loop(0, _W, fill_d, 0)

        base = d * (_H * _W)

        for p in range(2):
            fill_h(p, blks[p])
            pltpu.async_copy(
                blks[p], out_hbm.at[pl.ds(base + p * _W, _W)], sems[p])

        def pipe(i, carry):
            for p in range(2):
                h = 2 * i + p
                pltpu.make_async_copy(
                    blks[p], out_hbm.at[pl.ds(base, _W)], sems[p]).wait()
                fill_h(h, blks[p])
                pltpu.async_copy(
                    blks[p], out_hbm.at[pl.ds(base + h * _W, _W)], sems[p])
            return carry
        lax.fori_loop(1, _H // 2, pipe, 0)

        for p in range(2):
            pltpu.make_async_copy(
                blks[p], out_hbm.at[pl.ds(base, _W)], sems[p]).wait()


def kernel(depth, height, width, batch_size, d_emb, h_emb, w_emb):
    mesh = plsc.VectorSubcoreMesh(core_axis_name="c", subcore_axis_name="s")
    out = pl.kernel(
        _body,
        out_type=jax.ShapeDtypeStruct((_D * _H * _W, _ROW), jnp.float32),
        mesh=mesh,
        scratch_types=[
            pltpu.VMEM((_D, _EMB), jnp.float32),
            pltpu.VMEM((_H, _EMB), jnp.float32),
            pltpu.VMEM((_W, _EMB), jnp.float32),
            pltpu.VMEM((_W, _ROW), jnp.float32),
            pltpu.VMEM((_W, _ROW), jnp.float32),
            pltpu.SemaphoreType.DMA,
            pltpu.SemaphoreType.DMA,
        ],
    )(d_emb, h_emb, w_emb)
    return out.reshape(1, _D * _H * _W, _ROW)
